# baseline (speedup 1.0000x reference)
import functools

import jax
import jax.numpy as jnp
from jax import lax
from jax.experimental import pallas as pl
from jax.experimental.pallas import tpu as pltpu
from jax.experimental.pallas import tpu_sc as plsc

B, S, H, E, TOP_K = 4, 8192, 2048, 64, 8

NC, NS, L = 2, 16, 16
NW = NC * NS
TPB = NW // B

S_SC = 512
R = S_SC // TPB
RC = 16
NCHUNK = R // RC

A = S - S_SC
CHUNK = 512
J = A // CHUNK


def _acc_chunk(buf_ref, acc_ref):
    @plsc.parallel_loop(0, H // L, 1, unroll=4)
    def _(v):
        sl = pl.ds(v * L, L)
        vals = [buf_ref[r, sl] for r in range(RC)]
        while len(vals) > 1:
            nxt = [vals[i] + vals[i + 1] for i in range(0, len(vals) - 1, 2)]
            if len(vals) % 2:
                nxt.append(vals[-1])
            vals = nxt
        acc_ref[sl] = acc_ref[sl] + vals[0]


def _sc_body(x_hbm, out_hbm, buf0, buf1, acc, sem0, sem1):
    wid = lax.axis_index("s") * NC + lax.axis_index("c")
    b = wid // TPB
    k = wid % TPB
    row0 = (S - S_SC) + k * R

    def zbody(v, _):
        acc[pl.ds(v * L, L)] = jnp.zeros((L,), jnp.float32)
        return 0
    lax.fori_loop(0, H // L, zbody, 0)

    def copy_in(c, buf, sem):
        return pltpu.make_async_copy(
            x_hbm.at[b, pl.ds(row0 + c * RC, RC), :], buf, sem)

    copy_in(0, buf0, sem0).start()

    def cbody(i, _):
        c0 = 2 * i
        copy_in(c0, buf0, sem0).wait()
        copy_in(c0 + 1, buf1, sem1).start()
        _acc_chunk(buf0, acc)
        copy_in(c0 + 1, buf1, sem1).wait()

        @pl.when(i < NCHUNK // 2 - 1)
        def _():
            copy_in(c0 + 2, buf0, sem0).start()
        _acc_chunk(buf1, acc)
        return 0
    lax.fori_loop(0, NCHUNK // 2, cbody, 0)

    pltpu.sync_copy(acc, out_hbm.at[k * B + b])


_sc_sum = functools.partial(
    pl.kernel,
    out_type=jax.ShapeDtypeStruct((NW, H), jnp.float32),
    mesh=plsc.VectorSubcoreMesh(core_axis_name="c", subcore_axis_name="s"),
    scratch_types=[
        pltpu.VMEM((RC, H), jnp.float32),
        pltpu.VMEM((RC, H), jnp.float32),
        pltpu.VMEM((H,), jnp.float32),
        pltpu.SemaphoreType.DMA,
        pltpu.SemaphoreType.DMA,
    ],
)(_sc_body)


def _tc_body(x_ref, out_ref):
    j = pl.program_id(1)

    @pl.when(j == 0)
    def _():
        out_ref[...] = jnp.zeros_like(out_ref)

    out_ref[0] += jnp.sum(x_ref[0], axis=0, keepdims=True)


def _tc_partial(hidden_states):
    return pl.pallas_call(
        _tc_body,
        grid=(B, J),
        in_specs=[pl.BlockSpec((1, CHUNK, H), lambda b, j: (b, j, 0))],
        out_specs=pl.BlockSpec((1, 1, H), lambda b, j: (b, 0, 0)),
        out_shape=jax.ShapeDtypeStruct((B, 1, H), jnp.float32),
        compiler_params=pltpu.CompilerParams(
            dimension_semantics=("arbitrary", "arbitrary"),
        ),
    )(hidden_states)


def _join_body(tc_ref, sc_ref, fc1w_ref, fc1b_ref, fc2w_ref, fc2b_ref,
               idx_ref, w_ref):
    total = tc_ref[...]
    for k in range(TPB):
        total = total + sc_ref[pl.ds(k * B, B), :]
    seq = total * (1.0 / S)
    x = jnp.dot(seq, fc1w_ref[...],
                preferred_element_type=jnp.float32) + fc1b_ref[...]
    x = 0.5 * x * (1.0 + lax.erf(x * 0.7071067811865476))
    logits = jnp.dot(x, fc2w_ref[...],
                     preferred_element_type=jnp.float32) + fc2b_ref[...]
    m = jnp.max(logits, axis=1, keepdims=True)
    e = jnp.exp(logits - m)
    probs = e / jnp.sum(e, axis=1, keepdims=True)

    iota = lax.broadcasted_iota(jnp.int32, (B, E), 1)
    neg = jnp.float32(-jnp.inf)
    p = probs
    vals, idxs = [], []
    for _ in range(TOP_K):
        mv = jnp.max(p, axis=1, keepdims=True)
        first = jnp.min(jnp.where(p >= mv, iota, E), axis=1, keepdims=True)
        vals.append(mv)
        idxs.append(first)
        p = jnp.where(iota == first, neg, p)
    topv = jnp.concatenate(vals, axis=1)
    topi = jnp.concatenate(idxs, axis=1)
    ew = jnp.exp(topv - topv[:, :1])
    w = ew / jnp.sum(ew, axis=1, keepdims=True)
    idx_ref[...] = topi
    w_ref[...] = w


def _join(tc_part, sc_part, fc1_w, fc1_b, fc2_w, fc2_b):
    return pl.pallas_call(
        _join_body,
        out_shape=[
            jax.ShapeDtypeStruct((B, TOP_K), jnp.int32),
            jax.ShapeDtypeStruct((B, TOP_K), jnp.float32),
        ],
    )(tc_part, sc_part, fc1_w, fc1_b.reshape(1, H), fc2_w,
      fc2_b.reshape(1, E))


def kernel(hidden_states, fc1_w, fc1_b, fc2_w, fc2_b):
    tc_part = _tc_partial(hidden_states).reshape(B, H)
    sc_part = _sc_sum(hidden_states)
    topk_idx, topk_weight = _join(tc_part, sc_part, fc1_w, fc1_b,
                                  fc2_w, fc2_b)
    return (topk_idx, topk_weight, jnp.float32(0.0))


# speedup vs baseline: 1.2758x; 1.2758x over previous
import functools

import jax
import jax.numpy as jnp
from jax import lax
from jax.experimental import pallas as pl
from jax.experimental.pallas import tpu as pltpu

B, S, H, E, TOP_K = 4, 8192, 2048, 64, 8
CHUNK = 1024
J = S // CHUNK


def _gate_body(x_ref, fc1w_hbm, fc1b_ref, fc2t_hbm, fc2b_ref,
               idx_ref, w_ref, acc_ref, fc1w_v, fc2t_v, wsem):
    b = pl.program_id(0)
    j = pl.program_id(1)

    @pl.when((b == 0) & (j == 0))
    def _init():
        acc_ref[...] = jnp.zeros_like(acc_ref)

    @pl.when((b == 0) & (j == 1))
    def _prefetch_weights():
        pltpu.make_async_copy(fc1w_hbm, fc1w_v, wsem).start()
        pltpu.make_async_copy(fc2t_hbm, fc2t_v, wsem).start()

    acc_ref[pl.ds(b, 1), :] += jnp.sum(x_ref[0], axis=0, keepdims=True)

    @pl.when((b == B - 1) & (j == J - 1))
    def _final():
        pltpu.make_async_copy(fc1w_hbm, fc1w_v, wsem).wait()
        pltpu.make_async_copy(fc2t_hbm, fc2t_v, wsem).wait()
        seq = acc_ref[...] * (1.0 / S)
        x = jnp.dot(seq, fc1w_v[...],
                    preferred_element_type=jnp.float32) + fc1b_ref[...]
        x = 0.5 * x * (1.0 + lax.erf(x * 0.7071067811865476))
        logits = lax.dot_general(
            x, fc2t_v[...], (((1,), (1,)), ((), ())),
            preferred_element_type=jnp.float32) + fc2b_ref[...]
        m = jnp.max(logits, axis=1, keepdims=True)
        e = jnp.exp(logits - m)
        probs = e / jnp.sum(e, axis=1, keepdims=True)

        iota = lax.broadcasted_iota(jnp.int32, (B, E), 1)
        neg = jnp.float32(-jnp.inf)
        p = probs
        vals, idxs = [], []
        for _ in range(TOP_K):
            mv = jnp.max(p, axis=1, keepdims=True)
            first = jnp.min(jnp.where(p >= mv, iota, E), axis=1,
                            keepdims=True)
            vals.append(mv)
            idxs.append(first)
            p = jnp.where(iota == first, neg, p)
        topv = jnp.concatenate(vals, axis=1)
        topi = jnp.concatenate(idxs, axis=1)
        ew = jnp.exp(topv - topv[:, :1])
        w = ew / jnp.sum(ew, axis=1, keepdims=True)
        idx_ref[...] = topi
        w_ref[...] = w


def _gate(hidden_states, fc1_w, fc1_b, fc2_t, fc2_b):
    return pl.pallas_call(
        _gate_body,
        grid=(B, J),
        in_specs=[
            pl.BlockSpec((1, CHUNK, H), lambda b, j: (b, j, 0)),
            pl.BlockSpec(memory_space=pl.ANY),
            pl.BlockSpec((1, H), lambda b, j: (0, 0)),
            pl.BlockSpec(memory_space=pl.ANY),
            pl.BlockSpec((1, E), lambda b, j: (0, 0)),
        ],
        out_specs=[
            pl.BlockSpec((B, TOP_K), lambda b, j: (0, 0)),
            pl.BlockSpec((B, TOP_K), lambda b, j: (0, 0)),
        ],
        out_shape=[
            jax.ShapeDtypeStruct((B, TOP_K), jnp.int32),
            jax.ShapeDtypeStruct((B, TOP_K), jnp.float32),
        ],
        scratch_shapes=[
            pltpu.VMEM((B, H), jnp.float32),
            pltpu.VMEM((H, H), jnp.float32),
            pltpu.VMEM((E, H), jnp.float32),
            pltpu.SemaphoreType.DMA,
        ],
        compiler_params=pltpu.CompilerParams(
            dimension_semantics=("arbitrary", "arbitrary"),
        ),
    )(hidden_states, fc1_w, fc1_b.reshape(1, H), fc2_t, fc2_b.reshape(1, E))


def kernel(hidden_states, fc1_w, fc1_b, fc2_w, fc2_b):
    topk_idx, topk_weight = _gate(hidden_states, fc1_w, fc1_b,
                                  fc2_w.T, fc2_b)
    return (topk_idx, topk_weight, jnp.float32(0.0))
